# baseline (speedup 1.0000x reference)
import functools

import jax
import jax.numpy as jnp
from jax import lax
from jax.experimental import pallas as pl
from jax.experimental.pallas import tpu as pltpu
from jax.experimental.pallas import tpu_sc as plsc

B, H, S, D = 8, 16, 2048, 128
P = 1024
NBH = B * H
NC, NS = 2, 16
NW = NC * NS
JOBS = NBH // NW
L = 16
CH = 128
NB = 4
LA = 2
CPJ = P // CH
NCH = 2 * JOBS * CPJ


def _chunk(refs, wid, i):
    kc, vc, ko, vo, kv_v, vv_v = refs
    job, c = divmod(i, CPJ)
    bh = wid * JOBS + job % JOBS
    src, dst, val = (kc, ko, kv_v) if job < JOBS else (vc, vo, vv_v)
    return (src.at[pl.ds(bh * S + c * CH, CH), :],
            dst.at[pl.ds(bh * P + c * CH, CH), :],
            val, job % JOBS, c)


def _merge_val(buf, val, jrow, c, cpos, rloc):
    @pl.when(cpos == c)
    def _():
        for v in range(D // L):
            buf[rloc, pl.ds(v * L, L)] = val[jrow, pl.ds(v * L, L)]


def _body(kc, vc, kv, vv, pidx, ko, vo, b0, b1, b2, b3, kv_v, vv_v, p_v,
          gsem, ssem):
    wid = lax.axis_index("s") * NC + lax.axis_index("c")
    bufs = [b0, b1, b2, b3]

    pltpu.sync_copy(pidx, p_v)
    pltpu.sync_copy(kv.at[pl.ds(wid * JOBS, JOBS), :], kv_v)
    pltpu.sync_copy(vv.at[pl.ds(wid * JOBS, JOBS), :], vv_v)
    ploc = p_v[...][0]
    cpos = ploc // CH
    rloc = ploc - cpos * CH

    refs = (kc, vc, ko, vo, kv_v, vv_v)
    gd = [None] * NCH
    sd = [None] * NCH
    for i in range(NCH + LA):
        if i < NCH:
            if i >= NB:
                sd[i - NB].wait()
            src, _, _, _, _ = _chunk(refs, wid, i)
            gd[i] = pltpu.async_copy(src, bufs[i % NB], gsem.at[i % NB])
        j = i - LA
        if 0 <= j < NCH:
            gd[j].wait()
            _, dst, val, jrow, c = _chunk(refs, wid, j)
            _merge_val(bufs[j % NB], val, jrow, c, cpos, rloc)
            sd[j] = pltpu.async_copy(bufs[j % NB], dst, ssem.at[j % NB])
    for j in range(NCH - NB, NCH):
        sd[j].wait()


@jax.jit
def _run(kc, vc, kv, vv, pidx):
    mesh = plsc.VectorSubcoreMesh(core_axis_name="c", subcore_axis_name="s")
    f = functools.partial(
        pl.kernel,
        out_type=[jax.ShapeDtypeStruct((NBH * P, D), jnp.float32)] * 2,
        mesh=mesh,
        scratch_types=[
            pltpu.VMEM((CH, D), jnp.float32),
            pltpu.VMEM((CH, D), jnp.float32),
            pltpu.VMEM((CH, D), jnp.float32),
            pltpu.VMEM((CH, D), jnp.float32),
            pltpu.VMEM((JOBS, D), jnp.float32),
            pltpu.VMEM((JOBS, D), jnp.float32),
            pltpu.VMEM((L,), jnp.int32),
            pltpu.SemaphoreType.DMA((NB,)),
            pltpu.SemaphoreType.DMA((NB,)),
        ],
    )(_body)
    return f(kc, vc, kv, vv, pidx)


def kernel(k_cache, v_cache, k_val, v_val, input_pos):
    kc = k_cache.reshape(NBH * S, D)
    vc = v_cache.reshape(NBH * S, D)
    kv = k_val.reshape(NBH, D)
    vv = v_val.reshape(NBH, D)
    pos = jnp.asarray(input_pos, jnp.int32)
    pidx = jnp.zeros((L,), jnp.int32).at[0].set(pos - 1)
    ko, vo = _run(kc, vc, kv, vv, pidx)
    return ko.reshape(B, H, P, D), vo.reshape(B, H, P, D)


# speedup vs baseline: 1.0221x; 1.0221x over previous
import functools

import jax
import jax.numpy as jnp
from jax import lax
from jax.experimental import pallas as pl
from jax.experimental.pallas import tpu as pltpu
from jax.experimental.pallas import tpu_sc as plsc

B, H, S, D = 8, 16, 2048, 128
P = 1024
NBH = B * H
NC, NS = 2, 16
NW = NC * NS
JOBS = NBH // NW
L = 16
CH = 256
NB = 3
LA = 2
CPJ = P // CH
NCH = 2 * JOBS * CPJ


def _chunk(refs, wid, i):
    kc, vc, ko, vo, kv_v, vv_v = refs
    job, c = divmod(i, CPJ)
    bh = wid * JOBS + job % JOBS
    src, dst, val = (kc, ko, kv_v) if job < JOBS else (vc, vo, vv_v)
    return (src.at[pl.ds(bh * S + c * CH, CH), :],
            dst.at[pl.ds(bh * P + c * CH, CH), :],
            val, job % JOBS, c)


def _merge_val(buf, val, jrow, c, cpos, rloc):
    @pl.when(cpos == c)
    def _():
        for v in range(D // L):
            buf[rloc, pl.ds(v * L, L)] = val[jrow, pl.ds(v * L, L)]


def _body(kc, vc, kv, vv, pidx, ko, vo, b0, b1, b2, kv_v, vv_v, p_v,
          gsem, ssem):
    wid = lax.axis_index("s") * NC + lax.axis_index("c")
    bufs = [b0, b1, b2]

    pltpu.sync_copy(pidx, p_v)
    pltpu.sync_copy(kv.at[pl.ds(wid * JOBS, JOBS), :], kv_v)
    pltpu.sync_copy(vv.at[pl.ds(wid * JOBS, JOBS), :], vv_v)
    ploc = p_v[...][0]
    cpos = ploc // CH
    rloc = ploc - cpos * CH

    refs = (kc, vc, ko, vo, kv_v, vv_v)
    gd = [None] * NCH
    sd = [None] * NCH
    for i in range(NCH + LA):
        if i < NCH:
            if i >= NB:
                sd[i - NB].wait()
            src, _, _, _, _ = _chunk(refs, wid, i)
            gd[i] = pltpu.async_copy(src, bufs[i % NB], gsem.at[i % NB])
        j = i - LA
        if 0 <= j < NCH:
            gd[j].wait()
            _, dst, val, jrow, c = _chunk(refs, wid, j)
            _merge_val(bufs[j % NB], val, jrow, c, cpos, rloc)
            sd[j] = pltpu.async_copy(bufs[j % NB], dst, ssem.at[j % NB])
    for j in range(NCH - NB, NCH):
        sd[j].wait()


@jax.jit
def _run(kc, vc, kv, vv, pidx):
    mesh = plsc.VectorSubcoreMesh(core_axis_name="c", subcore_axis_name="s")
    f = functools.partial(
        pl.kernel,
        out_type=[jax.ShapeDtypeStruct((NBH * P, D), jnp.float32)] * 2,
        mesh=mesh,
        scratch_types=[
            pltpu.VMEM((CH, D), jnp.float32),
            pltpu.VMEM((CH, D), jnp.float32),
            pltpu.VMEM((CH, D), jnp.float32),
            pltpu.VMEM((JOBS, D), jnp.float32),
            pltpu.VMEM((JOBS, D), jnp.float32),
            pltpu.VMEM((L,), jnp.int32),
            pltpu.SemaphoreType.DMA((NB,)),
            pltpu.SemaphoreType.DMA((NB,)),
        ],
    )(_body)
    return f(kc, vc, kv, vv, pidx)


def kernel(k_cache, v_cache, k_val, v_val, input_pos):
    kc = k_cache.reshape(NBH * S, D)
    vc = v_cache.reshape(NBH * S, D)
    kv = k_val.reshape(NBH, D)
    vv = v_val.reshape(NBH, D)
    pos = jnp.asarray(input_pos, jnp.int32)
    pidx = jnp.zeros((L,), jnp.int32).at[0].set(pos - 1)
    ko, vo = _run(kc, vc, kv, vv, pidx)
    return ko.reshape(B, H, P, D), vo.reshape(B, H, P, D)
